# baseline (speedup 1.0000x reference)
import functools

import jax
import jax.numpy as jnp
import numpy as np
from jax.experimental import pallas as pl
from jax.experimental.pallas import tpu as pltpu


def _upsample_matrix() -> np.ndarray:
    W = np.zeros((96, 48), np.float32)
    for j in range(96):
        c = j / 2 - 0.25
        k0 = int(np.floor(c))
        w1 = c - k0
        taps = [(k0, 1.0 - w1), (k0 + 1, w1)]
        valid = [(k, w) for k, w in taps if 0 <= k < 48]
        s = sum(w for _, w in valid)
        for k, w in valid:
            W[j, k] = w / s
    return W


_W96x48 = _upsample_matrix()

_BH1 = 8
_NI = 48 // _BH1


def _fused_body(v1_ref, v2_ref, v3_ref, w_ref, map_ref, vol_out_ref,
                loss_ref, m_s, acc_s):
    b = pl.program_id(0)
    i = pl.program_id(1)

    sums = (jnp.sum(v1_ref[0], axis=(-2, -1))
            + jnp.sum(v2_ref[0], axis=(-2, -1))
            + jnp.sum(v3_ref[0], axis=(-2, -1))) * (1.0 / 2304.0)
    m_s[pl.ds(i * _BH1, _BH1), :] = sums

    @pl.when(jnp.logical_and(b == 0, i == 0))
    def _():
        acc_s[0] = jnp.float32(0.0)

    @pl.when(i == _NI - 1)
    def _():
        w = w_ref[...]
        t = jax.lax.dot(w, m_s[...], precision=jax.lax.Precision.HIGHEST)
        up = jax.lax.dot_general(
            t, w, (((1,), (1,)), ((), ())),
            precision=jax.lax.Precision.HIGHEST)
        vol_out_ref[0] = up
        d = up - map_ref[0]
        acc_s[0] = acc_s[0] + jnp.sum(d * d)

        @pl.when(b == 7)
        def _():
            loss_ref[0, 0] = acc_s[0] * (1.0 / (8 * 96 * 96))


@jax.jit
def kernel(out_volume, out_map, label):
    del label

    wmat = jnp.asarray(_W96x48)
    map3 = out_map.reshape(8, 96, 96)

    out_vol, loss = pl.pallas_call(
        _fused_body,
        grid=(8, _NI),
        in_specs=[
            pl.BlockSpec((1, _BH1, 48, 16, 48),
                         lambda b, i: (b, i, 0, 0, 0)),
            pl.BlockSpec((1, _BH1, 48, 16, 48),
                         lambda b, i: (b, i, 0, 1, 0)),
            pl.BlockSpec((1, _BH1, 48, 16, 48),
                         lambda b, i: (b, i, 0, 2, 0)),
            pl.BlockSpec((96, 48), lambda b, i: (0, 0)),
            pl.BlockSpec((1, 96, 96), lambda b, i: (b, 0, 0)),
        ],
        out_specs=[
            pl.BlockSpec((1, 96, 96), lambda b, i: (b, 0, 0)),
            pl.BlockSpec(memory_space=pltpu.SMEM),
        ],
        out_shape=[
            jax.ShapeDtypeStruct((8, 96, 96), jnp.float32),
            jax.ShapeDtypeStruct((1, 1), jnp.float32),
        ],
        scratch_shapes=[
            pltpu.VMEM((48, 48), jnp.float32),
            pltpu.SMEM((1,), jnp.float32),
        ],
    )(out_volume, out_volume, out_volume, wmat, map3)

    return loss[0, 0], out_vol


# speedup vs baseline: 1.0394x; 1.0394x over previous
import functools

import jax
import jax.numpy as jnp
import numpy as np
from jax.experimental import pallas as pl
from jax.experimental.pallas import tpu as pltpu


def _upsample_matrix() -> np.ndarray:
    W = np.zeros((96, 48), np.float32)
    for j in range(96):
        c = j / 2 - 0.25
        k0 = int(np.floor(c))
        w1 = c - k0
        taps = [(k0, 1.0 - w1), (k0 + 1, w1)]
        valid = [(k, w) for k, w in taps if 0 <= k < 48]
        s = sum(w for _, w in valid)
        for k, w in valid:
            W[j, k] = w / s
    return W


_W96x48 = _upsample_matrix()

_BH1 = 16
_NI = 48 // _BH1


def _fused_body(v1_ref, v2_ref, w_ref, map_ref, vol_out_ref, loss_ref,
                m_s, acc_s):
    b = pl.program_id(0)
    i = pl.program_id(1)

    sums = (jnp.sum(v1_ref[0], axis=(-2, -1))
            + jnp.sum(v2_ref[0], axis=(-2, -1))) * (1.0 / 2304.0)
    m_s[pl.ds(i * _BH1, _BH1), :] = sums

    @pl.when(jnp.logical_and(b == 0, i == 0))
    def _():
        acc_s[0] = jnp.float32(0.0)

    @pl.when(i == _NI - 1)
    def _():
        w = w_ref[...]
        t = jax.lax.dot(w, m_s[...], precision=jax.lax.Precision.HIGHEST)
        up = jax.lax.dot_general(
            t, w, (((1,), (1,)), ((), ())),
            precision=jax.lax.Precision.HIGHEST)
        vol_out_ref[0] = up
        d = up - map_ref[0]
        acc_s[0] = acc_s[0] + jnp.sum(d * d)

        @pl.when(b == 7)
        def _():
            loss_ref[0, 0] = acc_s[0] * (1.0 / (8 * 96 * 96))


@jax.jit
def kernel(out_volume, out_map, label):
    del label

    wmat = jnp.asarray(_W96x48)
    map3 = out_map.reshape(8, 96, 96)

    out_vol, loss = pl.pallas_call(
        _fused_body,
        grid=(8, _NI),
        in_specs=[
            pl.BlockSpec((1, _BH1, 48, 24, 48),
                         lambda b, i: (b, i, 0, 0, 0)),
            pl.BlockSpec((1, _BH1, 48, 24, 48),
                         lambda b, i: (b, i, 0, 1, 0)),
            pl.BlockSpec((96, 48), lambda b, i: (0, 0)),
            pl.BlockSpec((1, 96, 96), lambda b, i: (b, 0, 0)),
        ],
        out_specs=[
            pl.BlockSpec((1, 96, 96), lambda b, i: (b, 0, 0)),
            pl.BlockSpec(memory_space=pltpu.SMEM),
        ],
        out_shape=[
            jax.ShapeDtypeStruct((8, 96, 96), jnp.float32),
            jax.ShapeDtypeStruct((1, 1), jnp.float32),
        ],
        scratch_shapes=[
            pltpu.VMEM((48, 48), jnp.float32),
            pltpu.SMEM((1,), jnp.float32),
        ],
    )(out_volume, out_volume, wmat, map3)

    return loss[0, 0], out_vol
